# Initial kernel scaffold; baseline (speedup 1.0000x reference)
import jax
import jax.numpy as jnp
from jax.experimental import pallas as pl


def kernel(inputs, edge_index, W0, b0, W1, b1, W2, b2):
    raise NotImplementedError("write your pallas kernel here")


# speedup vs baseline: 8.1167x; 8.1167x over previous
import functools

import jax
import jax.numpy as jnp
from jax import lax
from jax.experimental import pallas as pl
from jax.experimental.pallas import tpu as pltpu
from jax.experimental.pallas import tpu_sc as plsc

N = 10000
E = 320000
D_IN = 128
HID = 256
NCLS = 64

NC = 2
NS = 16
K = 125
NP = 10240
RPT = NP // NS
KZ = 128
ZC = RPT // KZ

_MESH = plsc.VectorSubcoreMesh(core_axis_name="c", subcore_axis_name="s")


GI = 8
DW = 128


def _deg_body(idx_hbm, zeros_hbm, ones_hbm, out_hbm, idxv, buf, acc):
    ci = lax.axis_index("c")
    si = lax.axis_index("s")
    pltpu.sync_copy(zeros_hbm, buf)

    @pl.loop(0, ZC)
    def _zero(i):
        pltpu.sync_copy(buf, acc.at[pl.ds(si * RPT + i * KZ, KZ)])

    pltpu.sync_copy(ones_hbm, buf)
    idx_t = idx_hbm.at[ci].at[si]
    ones_k = buf.at[pl.ds(0, K)]
    plsc.subcore_barrier()

    @pl.loop(0, E // NS // K // GI)
    def _group(g):
        pltpu.sync_copy(idx_t.at[pl.ds(g * GI, GI)], idxv)
        for b in range(GI):
            pltpu.sync_copy(ones_k, acc.at[idxv.at[b]], add=True)

    plsc.subcore_barrier()
    pltpu.sync_copy(acc.at[pl.ds(si * RPT, RPT)],
                    out_hbm.at[ci].at[pl.ds(si * RPT, RPT)])


_deg_call = pl.kernel(
    _deg_body,
    out_type=jax.ShapeDtypeStruct((NC, NP, DW), jnp.float32),
    mesh=_MESH,
    scratch_types=[
        pltpu.VMEM((GI, K), jnp.int32),
        pltpu.VMEM((KZ, DW), jnp.float32),
        pltpu.VMEM_SHARED((NP, DW), jnp.float32),
    ],
)


def _agg_body(nchunks, edge_split, h_hbm, src_hbm, dst_hbm, zeros_hbm, out_hbm,
              srcv, dstv, rows0, rows1, acc, sem0, sem1):
    ci = lax.axis_index("c")
    si = lax.axis_index("s")
    pltpu.sync_copy(zeros_hbm, rows0)

    @pl.loop(0, ZC)
    def _zero(i):
        pltpu.sync_copy(rows0, acc.at[pl.ds(si * RPT + i * KZ, KZ)])

    wid = ci * NS + si if edge_split else si
    src_w = src_hbm.at[wid]
    dst_w = dst_hbm.at[wid]
    hsel = h_hbm if edge_split else h_hbm.at[ci]
    plsc.subcore_barrier()

    bufs = (rows0.at[pl.ds(0, K)], rows1.at[pl.ds(0, K)])
    sems = (sem0, sem1)

    @pl.loop(0, nchunks // GI)
    def _group(g):
        pltpu.sync_copy(src_w.at[pl.ds(g * GI, GI)], srcv)
        pltpu.sync_copy(dst_w.at[pl.ds(g * GI, GI)], dstv)
        pltpu.async_copy(hsel.at[srcv.at[0]], bufs[0], sems[0])
        for b in range(GI):
            cur = b % 2
            pltpu.make_async_copy(hsel.at[srcv.at[b]], bufs[cur], sems[cur]).wait()
            if b + 1 < GI:
                pltpu.async_copy(hsel.at[srcv.at[b + 1]], bufs[1 - cur], sems[1 - cur])
            pltpu.sync_copy(bufs[cur], acc.at[dstv.at[b]], add=True)

    plsc.subcore_barrier()
    pltpu.sync_copy(acc.at[pl.ds(si * RPT, RPT)],
                    out_hbm.at[ci].at[pl.ds(si * RPT, RPT)])


def _make_agg(c, edge_split):
    nchunks = E // (NC * NS if edge_split else NS) // K
    return pl.kernel(
        functools.partial(_agg_body, nchunks, edge_split),
        out_type=jax.ShapeDtypeStruct((NC, NP, c), jnp.float32),
        mesh=_MESH,
        scratch_types=[
            pltpu.VMEM((GI, K), jnp.int32),
            pltpu.VMEM((GI, K), jnp.int32),
            pltpu.VMEM((KZ, c), jnp.float32),
            pltpu.VMEM((KZ, c), jnp.float32),
            pltpu.VMEM_SHARED((NP, c), jnp.float32),
            pltpu.SemaphoreType.DMA,
            pltpu.SemaphoreType.DMA,
        ],
    )


_agg_e128 = _make_agg(D_IN, True)
_agg_c128 = _make_agg(HID // 2, False)


BM = 1000
GRID = N // BM


def _nrm(deg):
    return lax.rsqrt(jnp.maximum(deg, 1.0))


def _tc_pre_body(x_ref, ds_ref, o_ref):
    o_ref[...] = x_ref[...] * _nrm(ds_ref[...])


def _tc_mm0_body(p_ref, dd_ref, ds_ref, w_ref, b_ref, o_ref):
    agg = p_ref[0] + p_ref[1]
    h = jnp.dot(agg, w_ref[...], preferred_element_type=jnp.float32)
    h = h * _nrm(dd_ref[...]) + b_ref[...]
    h = jnp.maximum(h, 0.0) * _nrm(ds_ref[...])
    o_ref[0] = h[:, :HID // 2]
    o_ref[1] = h[:, HID // 2:]


def _tc_mm1_body(a_ref, dd_ref, ds_ref, w1_ref, b1_ref, w2_ref, o_ref):
    agg = jnp.concatenate([a_ref[0], a_ref[1]], axis=-1)
    h = jnp.dot(agg, w1_ref[...], preferred_element_type=jnp.float32)
    h = h * _nrm(dd_ref[...]) + b1_ref[...]
    h = jnp.maximum(h, 0.0) * _nrm(ds_ref[...])
    m2 = jnp.dot(h, w2_ref[...], preferred_element_type=jnp.float32)
    o_ref[...] = jnp.concatenate(
        [m2, jnp.zeros((BM, D_IN - NCLS), jnp.float32)], axis=-1)


def _tc_post_body(a_ref, dd_ref, b_ref, o_ref):
    agg = a_ref[0, :, :NCLS] + a_ref[1, :, :NCLS]
    o_ref[...] = agg * _nrm(dd_ref[...]) + b_ref[...]


def _row_spec(w):
    return pl.BlockSpec((BM, w), lambda i: (i, 0))


def _half_spec(w):
    return pl.BlockSpec((NC, BM, w), lambda i: (0, i, 0))


def _full_spec(shape):
    nd = len(shape)
    return pl.BlockSpec(shape, lambda i, _n=nd: (0,) * _n)


_tc_pre = pl.pallas_call(
    _tc_pre_body,
    grid=(GRID,),
    in_specs=[_row_spec(D_IN), _row_spec(1)],
    out_specs=_row_spec(D_IN),
    out_shape=jax.ShapeDtypeStruct((N, D_IN), jnp.float32),
)

_tc_mm0 = pl.pallas_call(
    _tc_mm0_body,
    grid=(GRID,),
    in_specs=[_half_spec(D_IN), _row_spec(1), _row_spec(1),
              _full_spec((D_IN, HID)), _full_spec((1, HID))],
    out_specs=_half_spec(HID // 2),
    out_shape=jax.ShapeDtypeStruct((NC, N, HID // 2), jnp.float32),
)

_tc_mm1 = pl.pallas_call(
    _tc_mm1_body,
    grid=(GRID,),
    in_specs=[_half_spec(HID // 2), _row_spec(1), _row_spec(1),
              _full_spec((HID, HID)), _full_spec((1, HID)),
              _full_spec((HID, NCLS))],
    out_specs=_row_spec(D_IN),
    out_shape=jax.ShapeDtypeStruct((N, D_IN), jnp.float32),
)

_tc_post = pl.pallas_call(
    _tc_post_body,
    grid=(GRID,),
    in_specs=[_half_spec(D_IN), _row_spec(1), _full_spec((1, NCLS))],
    out_specs=_row_spec(NCLS),
    out_shape=jax.ShapeDtypeStruct((N, NCLS), jnp.float32),
)


def kernel(inputs, edge_index, W0, b0, W1, b1, W2, b2):
    src = edge_index[0]
    dst = edge_index[1]
    idx_r = edge_index.reshape(2, NS, E // NS // K, K)
    src32 = src.reshape(NC * NS, E // (NC * NS) // K, K)
    dst32 = dst.reshape(NC * NS, E // (NC * NS) // K, K)
    src16 = src.reshape(NS, E // NS // K, K)
    dst16 = dst.reshape(NS, E // NS // K, K)

    zeros_128 = jnp.zeros((KZ, D_IN), jnp.float32)
    ones_128 = jnp.ones((KZ, DW), jnp.float32)

    deg = _deg_call(idx_r, zeros_128, ones_128)
    d_src = deg[0, :N, :1]
    d_dst = deg[1, :N, :1]

    xs = _tc_pre(inputs, d_src)
    p0 = _agg_e128(xs, src32, dst32, zeros_128)[:, :N]
    h2 = _tc_mm0(p0, d_dst, d_src, W0, b0.reshape(1, HID))
    a1 = _agg_c128(h2, src16, dst16, zeros_128)[:, :N]
    m2 = _tc_mm1(a1, d_dst, d_src, W1, b1.reshape(1, HID), W2)
    p2 = _agg_e128(m2, src32, dst32, zeros_128)[:, :N]
    out = _tc_post(p2, d_dst, b2.reshape(1, NCLS))
    return out


# speedup vs baseline: 8.4144x; 1.0367x over previous
import functools

import jax
import jax.numpy as jnp
from jax import lax
from jax.experimental import pallas as pl
from jax.experimental.pallas import tpu as pltpu
from jax.experimental.pallas import tpu_sc as plsc

N = 10000
E = 320000
D_IN = 128
HID = 256
NCLS = 64

NC = 2
NS = 16
K = 125
NP = 10240
RPT = NP // NS
KZ = 128
ZC = RPT // KZ

_MESH = plsc.VectorSubcoreMesh(core_axis_name="c", subcore_axis_name="s")


GI = 8
DW = 128


def _deg_body(idx_hbm, zeros_hbm, ones_hbm, out_hbm, idxv, buf, acc):
    ci = lax.axis_index("c")
    si = lax.axis_index("s")
    pltpu.sync_copy(zeros_hbm, buf)

    @pl.loop(0, ZC)
    def _zero(i):
        pltpu.sync_copy(buf, acc.at[pl.ds(si * RPT + i * KZ, KZ)])

    pltpu.sync_copy(ones_hbm, buf)
    idx_t = idx_hbm.at[ci].at[si]
    ones_k = buf.at[pl.ds(0, K)]
    plsc.subcore_barrier()

    @pl.loop(0, E // NS // K // GI)
    def _group(g):
        pltpu.sync_copy(idx_t.at[pl.ds(g * GI, GI)], idxv)
        for b in range(GI):
            pltpu.sync_copy(ones_k, acc.at[idxv.at[b]], add=True)

    plsc.subcore_barrier()
    pltpu.sync_copy(acc.at[pl.ds(si * RPT, RPT)],
                    out_hbm.at[ci].at[pl.ds(si * RPT, RPT)])


_deg_call = pl.kernel(
    _deg_body,
    out_type=jax.ShapeDtypeStruct((NC, NP, DW), jnp.float32),
    mesh=_MESH,
    scratch_types=[
        pltpu.VMEM((GI, K), jnp.int32),
        pltpu.VMEM((KZ, DW), jnp.float32),
        pltpu.VMEM_SHARED((NP, DW), jnp.float32),
    ],
)


def _agg_body(nchunks, edge_split, h_hbm, src_hbm, dst_hbm, zeros_hbm, out_hbm,
              srcv, dstv, rows0, rows1, acc, sem0, sem1):
    ci = lax.axis_index("c")
    si = lax.axis_index("s")
    pltpu.sync_copy(zeros_hbm, rows0)

    @pl.loop(0, ZC)
    def _zero(i):
        pltpu.sync_copy(rows0, acc.at[pl.ds(si * RPT + i * KZ, KZ)])

    wid = ci * NS + si if edge_split else si
    src_w = src_hbm.at[wid]
    dst_w = dst_hbm.at[wid]
    hsel = h_hbm if edge_split else h_hbm.at[ci]
    plsc.subcore_barrier()

    bufs = (rows0.at[pl.ds(0, K)], rows1.at[pl.ds(0, K)])
    sems = (sem0, sem1)

    @pl.loop(0, nchunks // GI)
    def _group(g):
        pltpu.sync_copy(src_w.at[pl.ds(g * GI, GI)], srcv)
        pltpu.sync_copy(dst_w.at[pl.ds(g * GI, GI)], dstv)
        pltpu.async_copy(hsel.at[srcv.at[0]], bufs[0], sems[0])
        for b in range(GI):
            cur = b % 2
            pltpu.make_async_copy(hsel.at[srcv.at[b]], bufs[cur], sems[cur]).wait()
            if b + 1 < GI:
                pltpu.async_copy(hsel.at[srcv.at[b + 1]], bufs[1 - cur], sems[1 - cur])
            pltpu.sync_copy(bufs[cur], acc.at[dstv.at[b]], add=True)

    plsc.subcore_barrier()
    pltpu.sync_copy(acc.at[pl.ds(si * RPT, RPT)],
                    out_hbm.at[ci].at[pl.ds(si * RPT, RPT)])


def _make_agg(c, edge_split):
    nchunks = E // (NC * NS if edge_split else NS) // K
    return pl.kernel(
        functools.partial(_agg_body, nchunks, edge_split),
        out_type=jax.ShapeDtypeStruct((NC, NP, c), jnp.float32),
        mesh=_MESH,
        scratch_types=[
            pltpu.VMEM((GI, K), jnp.int32),
            pltpu.VMEM((GI, K), jnp.int32),
            pltpu.VMEM((KZ, c), jnp.float32),
            pltpu.VMEM((KZ, c), jnp.float32),
            pltpu.VMEM_SHARED((NP, c), jnp.float32),
            pltpu.SemaphoreType.DMA,
            pltpu.SemaphoreType.DMA,
        ],
    )


_agg_e128 = _make_agg(D_IN, True)
_agg_c128 = _make_agg(HID // 2, False)


BM = 1000
GRID = N // BM


def _nrm(deg):
    return lax.rsqrt(jnp.maximum(deg, 1.0))


def _tc_pre_body(x_ref, deg_ref, o_ref):
    o_ref[...] = x_ref[...] * _nrm(deg_ref[0, :, :1])


def _tc_mm0_body(p_ref, degs_ref, degd_ref, w_ref, b_ref, o_ref):
    agg = p_ref[0] + p_ref[1]
    h = jnp.dot(agg, w_ref[...], preferred_element_type=jnp.float32)
    h = h * _nrm(degd_ref[0, :, :1]) + b_ref[...]
    h = jnp.maximum(h, 0.0) * _nrm(degs_ref[0, :, :1])
    o_ref[0] = h[:, :HID // 2]
    o_ref[1] = h[:, HID // 2:]


def _tc_mm1_body(a_ref, degs_ref, degd_ref, w1_ref, b1_ref, w2_ref, o_ref):
    agg = jnp.concatenate([a_ref[0], a_ref[1]], axis=-1)
    h = jnp.dot(agg, w1_ref[...], preferred_element_type=jnp.float32)
    h = h * _nrm(degd_ref[0, :, :1]) + b1_ref[...]
    h = jnp.maximum(h, 0.0) * _nrm(degs_ref[0, :, :1])
    m2 = jnp.dot(h, w2_ref[...], preferred_element_type=jnp.float32)
    o_ref[...] = jnp.concatenate(
        [m2, jnp.zeros((BM, D_IN - NCLS), jnp.float32)], axis=-1)


def _tc_post_body(a_ref, degd_ref, b_ref, o_ref):
    agg = a_ref[0, :, :NCLS] + a_ref[1, :, :NCLS]
    o_ref[...] = agg * _nrm(degd_ref[0, :, :1]) + b_ref[...]


def _row_spec(w):
    return pl.BlockSpec((BM, w), lambda i: (i, 0))


def _deg_spec(plane):
    return pl.BlockSpec((1, BM, DW), lambda i, _p=plane: (_p, i, 0))


def _half_spec(w):
    return pl.BlockSpec((NC, BM, w), lambda i: (0, i, 0))


def _full_spec(shape):
    nd = len(shape)
    return pl.BlockSpec(shape, lambda i, _n=nd: (0,) * _n)


_tc_pre = pl.pallas_call(
    _tc_pre_body,
    grid=(GRID,),
    in_specs=[_row_spec(D_IN), _deg_spec(0)],
    out_specs=_row_spec(D_IN),
    out_shape=jax.ShapeDtypeStruct((N, D_IN), jnp.float32),
)

_tc_mm0 = pl.pallas_call(
    _tc_mm0_body,
    grid=(GRID,),
    in_specs=[_half_spec(D_IN), _deg_spec(0), _deg_spec(1),
              _full_spec((D_IN, HID)), _full_spec((1, HID))],
    out_specs=_half_spec(HID // 2),
    out_shape=jax.ShapeDtypeStruct((NC, N, HID // 2), jnp.float32),
)

_tc_mm1 = pl.pallas_call(
    _tc_mm1_body,
    grid=(GRID,),
    in_specs=[_half_spec(HID // 2), _deg_spec(0), _deg_spec(1),
              _full_spec((HID, HID)), _full_spec((1, HID)),
              _full_spec((HID, NCLS))],
    out_specs=_row_spec(D_IN),
    out_shape=jax.ShapeDtypeStruct((N, D_IN), jnp.float32),
)

_tc_post = pl.pallas_call(
    _tc_post_body,
    grid=(GRID,),
    in_specs=[_half_spec(D_IN), _deg_spec(1), _full_spec((1, NCLS))],
    out_specs=_row_spec(NCLS),
    out_shape=jax.ShapeDtypeStruct((N, NCLS), jnp.float32),
)


def kernel(inputs, edge_index, W0, b0, W1, b1, W2, b2):
    src = edge_index[0]
    dst = edge_index[1]
    idx_r = edge_index.reshape(2, NS, E // NS // K, K)
    src32 = src.reshape(NC * NS, E // (NC * NS) // K, K)
    dst32 = dst.reshape(NC * NS, E // (NC * NS) // K, K)
    src16 = src.reshape(NS, E // NS // K, K)
    dst16 = dst.reshape(NS, E // NS // K, K)

    zeros_128 = jnp.zeros((KZ, D_IN), jnp.float32)
    ones_128 = jnp.ones((KZ, DW), jnp.float32)

    deg = _deg_call(idx_r, zeros_128, ones_128)

    xs = _tc_pre(inputs, deg)
    p0 = _agg_e128(xs, src32, dst32, zeros_128)
    h2 = _tc_mm0(p0, deg, deg, W0, b0.reshape(1, HID))
    a1 = _agg_c128(h2, src16, dst16, zeros_128)
    m2 = _tc_mm1(a1, deg, deg, W1, b1.reshape(1, HID), W2)
    p2 = _agg_e128(m2, src32, dst32, zeros_128)
    out = _tc_post(p2, deg, b2.reshape(1, NCLS))
    return out


# speedup vs baseline: 9.5199x; 1.1314x over previous
import functools

import jax
import jax.numpy as jnp
from jax import lax
from jax.experimental import pallas as pl
from jax.experimental.pallas import tpu as pltpu
from jax.experimental.pallas import tpu_sc as plsc

N = 10000
E = 320000
D_IN = 128
HID = 256
NCLS = 64

NC = 2
NS = 16
K = 125
NP = 10240
RPT = NP // NS
KZ = 128
ZC = RPT // KZ
GI = 8
DW = 128

_MESH = plsc.VectorSubcoreMesh(core_axis_name="c", subcore_axis_name="s")


def _deg_body(idx_hbm, zeros_hbm, ones_hbm, out_hbm, idxv, buf, acc, sem):
    ci = lax.axis_index("c")
    si = lax.axis_index("s")
    pltpu.sync_copy(zeros_hbm, buf)

    @pl.loop(0, ZC)
    def _zero(i):
        pltpu.sync_copy(buf, acc.at[pl.ds(si * RPT + i * KZ, KZ)])

    pltpu.sync_copy(ones_hbm, buf)
    idx_t = idx_hbm.at[ci].at[si]
    ones_k = buf.at[pl.ds(0, K)]
    plsc.subcore_barrier()

    @pl.loop(0, E // NS // K // GI)
    def _group(g):
        off = (g % 2) * GI
        pltpu.sync_copy(idx_t.at[pl.ds(g * GI, GI)], idxv.at[pl.ds(off, GI)])

        @pl.when(g > 0)
        def _drain():
            for _ in range(GI):
                pltpu.make_async_copy(ones_k, acc.at[idxv.at[0]], sem).wait()

        for b in range(GI):
            pltpu.make_async_copy(ones_k, acc.at[idxv.at[off + b]],
                                  sem).start(add=True)

    for _ in range(GI):
        pltpu.make_async_copy(ones_k, acc.at[idxv.at[0]], sem).wait()
    plsc.subcore_barrier()
    pltpu.sync_copy(acc.at[pl.ds(si * RPT, RPT)],
                    out_hbm.at[ci].at[pl.ds(si * RPT, RPT)])


_deg_call = pl.kernel(
    _deg_body,
    out_type=jax.ShapeDtypeStruct((NC, NP, DW), jnp.float32),
    mesh=_MESH,
    scratch_types=[
        pltpu.VMEM((2 * GI, K), jnp.int32),
        pltpu.VMEM((KZ, DW), jnp.float32),
        pltpu.VMEM_SHARED((NP, DW), jnp.float32),
        pltpu.SemaphoreType.DMA,
    ],
)


def _agg_body(nchunks, edge_split, h_hbm, src_hbm, dst_hbm, zeros_hbm, out_hbm,
              srcv, dstv, rows0, rows1, acc, sg0, sg1, ss0, ss1):
    ci = lax.axis_index("c")
    si = lax.axis_index("s")
    pltpu.sync_copy(zeros_hbm, rows0)

    @pl.loop(0, ZC)
    def _zero(i):
        pltpu.sync_copy(rows0, acc.at[pl.ds(si * RPT + i * KZ, KZ)])

    wid = ci * NS + si if edge_split else si
    src_w = src_hbm.at[wid]
    dst_w = dst_hbm.at[wid]
    hsel = h_hbm if edge_split else h_hbm.at[ci]
    plsc.subcore_barrier()

    bufs = (rows0.at[pl.ds(0, K)], rows1.at[pl.ds(0, K)])
    sgs = (sg0, sg1)
    sss = (ss0, ss1)

    def _gwait(b):
        pltpu.make_async_copy(hsel.at[srcv.at[0]], bufs[b % 2], sgs[b % 2]).wait()

    def _swait(b):
        pltpu.make_async_copy(bufs[b % 2], acc.at[dstv.at[0]], sss[b % 2]).wait()

    @pl.loop(0, nchunks // GI)
    def _group(g):
        off = (g % 2) * GI
        pltpu.sync_copy(src_w.at[pl.ds(g * GI, GI)], srcv)
        pltpu.sync_copy(dst_w.at[pl.ds(g * GI, GI)], dstv.at[pl.ds(off, GI)])

        @pl.when(g > 0)
        def _free0():
            _swait(0)

        pltpu.async_copy(hsel.at[srcv.at[0]], bufs[0], sgs[0])
        for b in range(GI):
            cur = b % 2
            nxt = 1 - cur
            if b + 1 < GI:
                if b == 0:
                    @pl.when(g > 0)
                    def _free1():
                        _swait(1)
                else:
                    _swait(b - 1)
                pltpu.async_copy(hsel.at[srcv.at[b + 1]], bufs[nxt], sgs[nxt])
            _gwait(b)
            pltpu.make_async_copy(bufs[cur], acc.at[dstv.at[off + b]],
                                  sss[cur]).start(add=True)

    _swait(0)
    _swait(1)
    plsc.subcore_barrier()
    pltpu.sync_copy(acc.at[pl.ds(si * RPT, RPT)],
                    out_hbm.at[ci].at[pl.ds(si * RPT, RPT)])


def _make_agg(c, edge_split):
    nchunks = E // (NC * NS if edge_split else NS) // K
    return pl.kernel(
        functools.partial(_agg_body, nchunks, edge_split),
        out_type=jax.ShapeDtypeStruct((NC, NP, c), jnp.float32),
        mesh=_MESH,
        scratch_types=[
            pltpu.VMEM((GI, K), jnp.int32),
            pltpu.VMEM((2 * GI, K), jnp.int32),
            pltpu.VMEM((KZ, c), jnp.float32),
            pltpu.VMEM((KZ, c), jnp.float32),
            pltpu.VMEM_SHARED((NP, c), jnp.float32),
            pltpu.SemaphoreType.DMA,
            pltpu.SemaphoreType.DMA,
            pltpu.SemaphoreType.DMA,
            pltpu.SemaphoreType.DMA,
        ],
    )


_agg_e128 = _make_agg(D_IN, True)
_agg_c128 = _make_agg(HID // 2, False)


BM = 1000
GRID = N // BM


def _nrm(deg):
    return lax.rsqrt(jnp.maximum(deg, 1.0))


def _tc_pre_body(x_ref, deg_ref, o_ref):
    o_ref[...] = x_ref[...] * _nrm(deg_ref[0, :, :1])


def _tc_mm0_body(p_ref, degs_ref, degd_ref, w_ref, b_ref, o_ref):
    agg = p_ref[0] + p_ref[1]
    h = jnp.dot(agg, w_ref[...], preferred_element_type=jnp.float32)
    h = h * _nrm(degd_ref[0, :, :1]) + b_ref[...]
    h = jnp.maximum(h, 0.0) * _nrm(degs_ref[0, :, :1])
    o_ref[0] = h[:, :HID // 2]
    o_ref[1] = h[:, HID // 2:]


def _tc_mm1_body(a_ref, degs_ref, degd_ref, w1_ref, b1_ref, w2_ref, o_ref):
    agg = jnp.concatenate([a_ref[0], a_ref[1]], axis=-1)
    h = jnp.dot(agg, w1_ref[...], preferred_element_type=jnp.float32)
    h = h * _nrm(degd_ref[0, :, :1]) + b1_ref[...]
    h = jnp.maximum(h, 0.0) * _nrm(degs_ref[0, :, :1])
    m2 = jnp.dot(h, w2_ref[...], preferred_element_type=jnp.float32)
    o_ref[...] = jnp.concatenate(
        [m2, jnp.zeros((BM, D_IN - NCLS), jnp.float32)], axis=-1)


def _tc_post_body(a_ref, degd_ref, b_ref, o_ref):
    agg = a_ref[0, :, :NCLS] + a_ref[1, :, :NCLS]
    o_ref[...] = agg * _nrm(degd_ref[0, :, :1]) + b_ref[...]


def _row_spec(w):
    return pl.BlockSpec((BM, w), lambda i: (i, 0))


def _deg_spec(plane):
    return pl.BlockSpec((1, BM, DW), lambda i, _p=plane: (_p, i, 0))


def _half_spec(w):
    return pl.BlockSpec((NC, BM, w), lambda i: (0, i, 0))


def _full_spec(shape):
    nd = len(shape)
    return pl.BlockSpec(shape, lambda i, _n=nd: (0,) * _n)


_tc_pre = pl.pallas_call(
    _tc_pre_body,
    grid=(GRID,),
    in_specs=[_row_spec(D_IN), _deg_spec(0)],
    out_specs=_row_spec(D_IN),
    out_shape=jax.ShapeDtypeStruct((N, D_IN), jnp.float32),
)

_tc_mm0 = pl.pallas_call(
    _tc_mm0_body,
    grid=(GRID,),
    in_specs=[_half_spec(D_IN), _deg_spec(0), _deg_spec(1),
              _full_spec((D_IN, HID)), _full_spec((1, HID))],
    out_specs=_half_spec(HID // 2),
    out_shape=jax.ShapeDtypeStruct((NC, N, HID // 2), jnp.float32),
)

_tc_mm1 = pl.pallas_call(
    _tc_mm1_body,
    grid=(GRID,),
    in_specs=[_half_spec(HID // 2), _deg_spec(0), _deg_spec(1),
              _full_spec((HID, HID)), _full_spec((1, HID)),
              _full_spec((HID, NCLS))],
    out_specs=_row_spec(D_IN),
    out_shape=jax.ShapeDtypeStruct((N, D_IN), jnp.float32),
)

_tc_post = pl.pallas_call(
    _tc_post_body,
    grid=(GRID,),
    in_specs=[_half_spec(D_IN), _deg_spec(1), _full_spec((1, NCLS))],
    out_specs=_row_spec(NCLS),
    out_shape=jax.ShapeDtypeStruct((N, NCLS), jnp.float32),
)


def kernel(inputs, edge_index, W0, b0, W1, b1, W2, b2):
    src = edge_index[0]
    dst = edge_index[1]
    idx_r = edge_index.reshape(2, NS, E // NS // K, K)
    src32 = src.reshape(NC * NS, E // (NC * NS) // K, K)
    dst32 = dst.reshape(NC * NS, E // (NC * NS) // K, K)
    src16 = src.reshape(NS, E // NS // K, K)
    dst16 = dst.reshape(NS, E // NS // K, K)

    zeros_128 = jnp.zeros((KZ, D_IN), jnp.float32)
    ones_128 = jnp.ones((KZ, DW), jnp.float32)

    deg = _deg_call(idx_r, zeros_128, ones_128)

    xs = _tc_pre(inputs, deg)
    p0 = _agg_e128(xs, src32, dst32, zeros_128)
    h2 = _tc_mm0(p0, deg, deg, W0, b0.reshape(1, HID))
    a1 = _agg_c128(h2, src16, dst16, zeros_128)
    m2 = _tc_mm1(a1, deg, deg, W1, b1.reshape(1, HID), W2)
    p2 = _agg_e128(m2, src32, dst32, zeros_128)
    out = _tc_post(p2, deg, b2.reshape(1, NCLS))
    return out


# speedup vs baseline: 10.4768x; 1.1005x over previous
import functools

import jax
import jax.numpy as jnp
from jax import lax
from jax.experimental import pallas as pl
from jax.experimental.pallas import tpu as pltpu
from jax.experimental.pallas import tpu_sc as plsc

N = 10000
E = 320000
D_IN = 128
HID = 256
NCLS = 64

NC = 2
NS = 16
K = 125
NP = 10240
RPT = NP // NS
KZ = 128
ZC = RPT // KZ
GI = 8
DW = 128

_MESH = plsc.VectorSubcoreMesh(core_axis_name="c", subcore_axis_name="s")


def _deg_body(idx_hbm, zeros_hbm, ones_hbm, out_hbm, idxv, buf, acc, sem):
    ci = lax.axis_index("c")
    si = lax.axis_index("s")
    pltpu.sync_copy(zeros_hbm, buf)

    @pl.loop(0, ZC)
    def _zero(i):
        pltpu.sync_copy(buf, acc.at[pl.ds(si * RPT + i * KZ, KZ)])

    pltpu.sync_copy(ones_hbm, buf)
    idx_t = idx_hbm.at[ci].at[si]
    ones_k = buf.at[pl.ds(0, K)]
    plsc.subcore_barrier()

    @pl.loop(0, E // NS // K // GI)
    def _group(g):
        off = (g % 2) * GI
        pltpu.sync_copy(idx_t.at[pl.ds(g * GI, GI)], idxv.at[pl.ds(off, GI)])

        @pl.when(g > 0)
        def _drain():
            for _ in range(GI):
                pltpu.make_async_copy(ones_k, acc.at[idxv.at[0]], sem).wait()

        for b in range(GI):
            pltpu.make_async_copy(ones_k, acc.at[idxv.at[off + b]],
                                  sem).start(add=True)

    for _ in range(GI):
        pltpu.make_async_copy(ones_k, acc.at[idxv.at[0]], sem).wait()
    plsc.subcore_barrier()
    pltpu.sync_copy(acc.at[pl.ds(si * RPT, RPT)],
                    out_hbm.at[ci].at[pl.ds(si * RPT, RPT)])


_deg_call = pl.kernel(
    _deg_body,
    out_type=jax.ShapeDtypeStruct((NC, NP, DW), jnp.float32),
    mesh=_MESH,
    scratch_types=[
        pltpu.VMEM((2 * GI, K), jnp.int32),
        pltpu.VMEM((KZ, DW), jnp.float32),
        pltpu.VMEM_SHARED((NP, DW), jnp.float32),
        pltpu.SemaphoreType.DMA,
    ],
)


def _agg_body(nchunks, edge_split, h_hbm, src_hbm, dst_hbm, zeros_hbm, out_hbm,
              srcv, dstv, rows0, rows1, acc, sg0, sg1, ss0, ss1, semi):
    ci = lax.axis_index("c")
    si = lax.axis_index("s")
    pltpu.sync_copy(zeros_hbm, rows0)

    @pl.loop(0, ZC)
    def _zero(i):
        pltpu.sync_copy(rows0, acc.at[pl.ds(si * RPT + i * KZ, KZ)])

    wid = ci * NS + si if edge_split else si
    src_w = src_hbm.at[wid]
    dst_w = dst_hbm.at[wid]
    hsel = h_hbm if edge_split else h_hbm.at[ci]
    plsc.subcore_barrier()

    bufs = (rows0.at[pl.ds(0, K)], rows1.at[pl.ds(0, K)])
    sgs = (sg0, sg1)
    sss = (ss0, ss1)

    def _gwait(b):
        pltpu.make_async_copy(hsel.at[srcv.at[0]], bufs[b % 2], sgs[b % 2]).wait()

    def _swait(b):
        pltpu.make_async_copy(bufs[b % 2], acc.at[dstv.at[0]], sss[b % 2]).wait()

    ngroups = nchunks // GI
    pltpu.sync_copy(src_w.at[pl.ds(0, GI)], srcv.at[pl.ds(0, GI)])
    pltpu.sync_copy(dst_w.at[pl.ds(0, GI)], dstv.at[pl.ds(0, GI)])

    @pl.loop(0, ngroups)
    def _group(g):
        soff = lax.rem(g, 2) * GI
        doff = lax.rem(g, 3) * GI
        sv = srcv.at[pl.ds(soff, GI)]

        @pl.when(g > 0)
        def _wait_stage():
            pltpu.make_async_copy(src_w.at[pl.ds(g * GI, GI)],
                                  srcv.at[pl.ds(soff, GI)], semi).wait()
            pltpu.make_async_copy(dst_w.at[pl.ds(g * GI, GI)],
                                  dstv.at[pl.ds(doff, GI)], semi).wait()

        @pl.when(g > 0)
        def _free0():
            _swait(0)

        pltpu.async_copy(hsel.at[sv.at[0]], bufs[0], sgs[0])

        @pl.when(g + 1 < ngroups)
        def _prefetch():
            nso = lax.rem(g + 1, 2) * GI
            ndo = lax.rem(g + 1, 3) * GI
            pltpu.async_copy(src_w.at[pl.ds((g + 1) * GI, GI)],
                             srcv.at[pl.ds(nso, GI)], semi)
            pltpu.async_copy(dst_w.at[pl.ds((g + 1) * GI, GI)],
                             dstv.at[pl.ds(ndo, GI)], semi)

        for b in range(GI):
            cur = b % 2
            nxt = 1 - cur
            if b + 1 < GI:
                if b == 0:
                    @pl.when(g > 0)
                    def _free1():
                        _swait(1)
                else:
                    _swait(b - 1)
                pltpu.async_copy(hsel.at[sv.at[b + 1]], bufs[nxt], sgs[nxt])
            _gwait(b)
            pltpu.make_async_copy(bufs[cur], acc.at[dstv.at[doff + b]],
                                  sss[cur]).start(add=True)

    _swait(0)
    _swait(1)
    plsc.subcore_barrier()
    pltpu.sync_copy(acc.at[pl.ds(si * RPT, RPT)],
                    out_hbm.at[ci].at[pl.ds(si * RPT, RPT)])


def _make_agg(c, edge_split):
    nchunks = E // (NC * NS if edge_split else NS) // K
    return pl.kernel(
        functools.partial(_agg_body, nchunks, edge_split),
        out_type=jax.ShapeDtypeStruct((NC, NP, c), jnp.float32),
        mesh=_MESH,
        scratch_types=[
            pltpu.VMEM((2 * GI, K), jnp.int32),
            pltpu.VMEM((3 * GI, K), jnp.int32),
            pltpu.VMEM((KZ, c), jnp.float32),
            pltpu.VMEM((KZ, c), jnp.float32),
            pltpu.VMEM_SHARED((NP, c), jnp.float32),
            pltpu.SemaphoreType.DMA,
            pltpu.SemaphoreType.DMA,
            pltpu.SemaphoreType.DMA,
            pltpu.SemaphoreType.DMA,
            pltpu.SemaphoreType.DMA,
        ],
    )


_agg_e128 = _make_agg(D_IN, True)
_agg_c128 = _make_agg(HID // 2, False)


BM = 2000
GRID = N // BM


def _nrm(deg):
    return lax.rsqrt(jnp.maximum(deg, 1.0))


def _tc_pre_body(x_ref, deg_ref, o_ref):
    o_ref[...] = x_ref[...] * _nrm(deg_ref[0, :, :1])


def _tc_mm0_body(p_ref, degs_ref, degd_ref, w_ref, b_ref, o_ref):
    agg = p_ref[0] + p_ref[1]
    h = jnp.dot(agg, w_ref[...], preferred_element_type=jnp.float32)
    h = h * _nrm(degd_ref[0, :, :1]) + b_ref[...]
    h = jnp.maximum(h, 0.0) * _nrm(degs_ref[0, :, :1])
    o_ref[0] = h[:, :HID // 2]
    o_ref[1] = h[:, HID // 2:]


def _tc_mm1_body(a_ref, degs_ref, degd_ref, w1_ref, b1_ref, w2_ref, o_ref):
    agg = jnp.concatenate([a_ref[0], a_ref[1]], axis=-1)
    h = jnp.dot(agg, w1_ref[...], preferred_element_type=jnp.float32)
    h = h * _nrm(degd_ref[0, :, :1]) + b1_ref[...]
    h = jnp.maximum(h, 0.0) * _nrm(degs_ref[0, :, :1])
    m2 = jnp.dot(h, w2_ref[...], preferred_element_type=jnp.float32)
    o_ref[...] = jnp.concatenate(
        [m2, jnp.zeros((BM, D_IN - NCLS), jnp.float32)], axis=-1)


def _tc_post_body(a_ref, degd_ref, b_ref, o_ref):
    agg = a_ref[0, :, :NCLS] + a_ref[1, :, :NCLS]
    o_ref[...] = agg * _nrm(degd_ref[0, :, :1]) + b_ref[...]


def _row_spec(w):
    return pl.BlockSpec((BM, w), lambda i: (i, 0))


def _deg_spec(plane):
    return pl.BlockSpec((1, BM, DW), lambda i, _p=plane: (_p, i, 0))


def _half_spec(w):
    return pl.BlockSpec((NC, BM, w), lambda i: (0, i, 0))


def _full_spec(shape):
    nd = len(shape)
    return pl.BlockSpec(shape, lambda i, _n=nd: (0,) * _n)


_tc_pre = pl.pallas_call(
    _tc_pre_body,
    grid=(GRID,),
    in_specs=[_row_spec(D_IN), _deg_spec(0)],
    out_specs=_row_spec(D_IN),
    out_shape=jax.ShapeDtypeStruct((N, D_IN), jnp.float32),
)

_tc_mm0 = pl.pallas_call(
    _tc_mm0_body,
    grid=(GRID,),
    in_specs=[_half_spec(D_IN), _deg_spec(0), _deg_spec(1),
              _full_spec((D_IN, HID)), _full_spec((1, HID))],
    out_specs=_half_spec(HID // 2),
    out_shape=jax.ShapeDtypeStruct((NC, N, HID // 2), jnp.float32),
)

_tc_mm1 = pl.pallas_call(
    _tc_mm1_body,
    grid=(GRID,),
    in_specs=[_half_spec(HID // 2), _deg_spec(0), _deg_spec(1),
              _full_spec((HID, HID)), _full_spec((1, HID)),
              _full_spec((HID, NCLS))],
    out_specs=_row_spec(D_IN),
    out_shape=jax.ShapeDtypeStruct((N, D_IN), jnp.float32),
)

_tc_post = pl.pallas_call(
    _tc_post_body,
    grid=(GRID,),
    in_specs=[_half_spec(D_IN), _deg_spec(1), _full_spec((1, NCLS))],
    out_specs=_row_spec(NCLS),
    out_shape=jax.ShapeDtypeStruct((N, NCLS), jnp.float32),
)


def kernel(inputs, edge_index, W0, b0, W1, b1, W2, b2):
    src = edge_index[0]
    dst = edge_index[1]
    idx_r = edge_index.reshape(2, NS, E // NS // K, K)
    src32 = src.reshape(NC * NS, E // (NC * NS) // K, K)
    dst32 = dst.reshape(NC * NS, E // (NC * NS) // K, K)
    src16 = src.reshape(NS, E // NS // K, K)
    dst16 = dst.reshape(NS, E // NS // K, K)

    zeros_128 = jnp.zeros((KZ, D_IN), jnp.float32)
    ones_128 = jnp.ones((KZ, DW), jnp.float32)

    deg = _deg_call(idx_r, zeros_128, ones_128)

    xs = _tc_pre(inputs, deg)
    p0 = _agg_e128(xs, src32, dst32, zeros_128)
    h2 = _tc_mm0(p0, deg, deg, W0, b0.reshape(1, HID))
    a1 = _agg_c128(h2, src16, dst16, zeros_128)
    m2 = _tc_mm1(a1, deg, deg, W1, b1.reshape(1, HID), W2)
    p2 = _agg_e128(m2, src32, dst32, zeros_128)
    out = _tc_post(p2, deg, b2.reshape(1, NCLS))
    return out


# speedup vs baseline: 10.4893x; 1.0012x over previous
import functools

import jax
import jax.numpy as jnp
from jax import lax
from jax.experimental import pallas as pl
from jax.experimental.pallas import tpu as pltpu
from jax.experimental.pallas import tpu_sc as plsc

N = 10000
E = 320000
D_IN = 128
HID = 256
NCLS = 64

NC = 2
NS = 16
K = 125
NP = 10240
RPT = NP // NS
KZ = 128
ZC = RPT // KZ
GI = 8
DW = 128

_MESH = plsc.VectorSubcoreMesh(core_axis_name="c", subcore_axis_name="s")


def _deg_body(idx_hbm, zeros_hbm, ones_hbm, out_hbm, idxv, buf, acc, sem):
    ci = lax.axis_index("c")
    si = lax.axis_index("s")
    pltpu.sync_copy(zeros_hbm, buf)

    @pl.loop(0, ZC)
    def _zero(i):
        pltpu.sync_copy(buf, acc.at[pl.ds(si * RPT + i * KZ, KZ)])

    pltpu.sync_copy(ones_hbm, buf)
    idx_t = idx_hbm.at[ci].at[si]
    ones_k = buf.at[pl.ds(0, K)]
    plsc.subcore_barrier()

    @pl.loop(0, E // NS // K // GI)
    def _group(g):
        off = (g % 2) * GI
        pltpu.sync_copy(idx_t.at[pl.ds(g * GI, GI)], idxv.at[pl.ds(off, GI)])

        @pl.when(g > 0)
        def _drain():
            for _ in range(GI):
                pltpu.make_async_copy(ones_k, acc.at[idxv.at[0]], sem).wait()

        for b in range(GI):
            pltpu.make_async_copy(ones_k, acc.at[idxv.at[off + b]],
                                  sem).start(add=True)

    for _ in range(GI):
        pltpu.make_async_copy(ones_k, acc.at[idxv.at[0]], sem).wait()
    plsc.subcore_barrier()
    pltpu.sync_copy(acc.at[pl.ds(si * RPT, RPT)],
                    out_hbm.at[ci].at[pl.ds(si * RPT, RPT)])


_deg_call = pl.kernel(
    _deg_body,
    out_type=jax.ShapeDtypeStruct((NC, NP, DW), jnp.float32),
    mesh=_MESH,
    scratch_types=[
        pltpu.VMEM((2 * GI, K), jnp.int32),
        pltpu.VMEM((KZ, DW), jnp.float32),
        pltpu.VMEM_SHARED((NP, DW), jnp.float32),
        pltpu.SemaphoreType.DMA,
    ],
)


def _agg_body(nchunks, edge_split, h_hbm, src_hbm, dst_hbm, zeros_hbm, out_hbm,
              srcv, dstv, rows0, rows1, acc, sg0, sg1, ss0, ss1, semi):
    ci = lax.axis_index("c")
    si = lax.axis_index("s")
    pltpu.sync_copy(zeros_hbm, rows0)

    @pl.loop(0, ZC)
    def _zero(i):
        pltpu.sync_copy(rows0, acc.at[pl.ds(si * RPT + i * KZ, KZ)])

    wid = ci * NS + si if edge_split else si
    src_w = src_hbm.at[wid]
    dst_w = dst_hbm.at[wid]
    hsel = h_hbm if edge_split else h_hbm.at[ci]
    plsc.subcore_barrier()

    bufs = (rows0.at[pl.ds(0, K)], rows1.at[pl.ds(0, K)])
    sgs = (sg0, sg1)
    sss = (ss0, ss1)

    def _gwait(b):
        pltpu.make_async_copy(hsel.at[srcv.at[0]], bufs[b % 2], sgs[b % 2]).wait()

    def _swait(b):
        pltpu.make_async_copy(bufs[b % 2], acc.at[dstv.at[0]], sss[b % 2]).wait()

    ngroups = nchunks // GI
    pltpu.sync_copy(src_w.at[pl.ds(0, GI)], srcv.at[pl.ds(0, GI)])
    pltpu.sync_copy(dst_w.at[pl.ds(0, GI)], dstv.at[pl.ds(0, GI)])

    pltpu.async_copy(hsel.at[srcv.at[0]], bufs[0], sgs[0])

    @pl.loop(0, ngroups)
    def _group(g):
        soff = lax.rem(g, 2) * GI
        doff = lax.rem(g, 3) * GI
        sv = srcv.at[pl.ds(soff, GI)]

        @pl.when(g + 1 < ngroups)
        def _prefetch():
            nso = lax.rem(g + 1, 2) * GI
            ndo = lax.rem(g + 1, 3) * GI
            pltpu.async_copy(src_w.at[pl.ds((g + 1) * GI, GI)],
                             srcv.at[pl.ds(nso, GI)], semi)
            pltpu.async_copy(dst_w.at[pl.ds((g + 1) * GI, GI)],
                             dstv.at[pl.ds(ndo, GI)], semi)

        for b in range(GI):
            cur = b % 2
            nxt = 1 - cur
            if b + 1 < GI:
                if b == 0:
                    @pl.when(g > 0)
                    def _free1():
                        _swait(1)
                else:
                    _swait(b - 1)
                pltpu.async_copy(hsel.at[sv.at[b + 1]], bufs[nxt], sgs[nxt])
            _gwait(b)
            pltpu.make_async_copy(bufs[cur], acc.at[dstv.at[doff + b]],
                                  sss[cur]).start(add=True)

        @pl.when(g + 1 < ngroups)
        def _head_next():
            nso = lax.rem(g + 1, 2) * GI
            ndo = lax.rem(g + 1, 3) * GI
            pltpu.make_async_copy(src_w.at[pl.ds((g + 1) * GI, GI)],
                                  srcv.at[pl.ds(nso, GI)], semi).wait()
            pltpu.make_async_copy(dst_w.at[pl.ds((g + 1) * GI, GI)],
                                  dstv.at[pl.ds(ndo, GI)], semi).wait()
            _swait(0)
            pltpu.async_copy(hsel.at[srcv.at[nso]], bufs[0], sgs[0])

    _swait(0)
    _swait(1)
    plsc.subcore_barrier()
    pltpu.sync_copy(acc.at[pl.ds(si * RPT, RPT)],
                    out_hbm.at[ci].at[pl.ds(si * RPT, RPT)])


def _make_agg(c, edge_split):
    nchunks = E // (NC * NS if edge_split else NS) // K
    return pl.kernel(
        functools.partial(_agg_body, nchunks, edge_split),
        out_type=jax.ShapeDtypeStruct((NC, NP, c), jnp.float32),
        mesh=_MESH,
        scratch_types=[
            pltpu.VMEM((2 * GI, K), jnp.int32),
            pltpu.VMEM((3 * GI, K), jnp.int32),
            pltpu.VMEM((KZ, c), jnp.float32),
            pltpu.VMEM((KZ, c), jnp.float32),
            pltpu.VMEM_SHARED((NP, c), jnp.float32),
            pltpu.SemaphoreType.DMA,
            pltpu.SemaphoreType.DMA,
            pltpu.SemaphoreType.DMA,
            pltpu.SemaphoreType.DMA,
            pltpu.SemaphoreType.DMA,
        ],
    )


_agg_e128 = _make_agg(D_IN, True)
_agg_c128 = _make_agg(HID // 2, False)


BM = 2000
GRID = N // BM


def _nrm(deg):
    return lax.rsqrt(jnp.maximum(deg, 1.0))


def _tc_pre_body(x_ref, deg_ref, o_ref):
    o_ref[...] = x_ref[...] * _nrm(deg_ref[0, :, :1])


def _tc_mm0_body(p_ref, degs_ref, degd_ref, w_ref, b_ref, o_ref):
    agg = p_ref[0] + p_ref[1]
    h = jnp.dot(agg, w_ref[...], preferred_element_type=jnp.float32)
    h = h * _nrm(degd_ref[0, :, :1]) + b_ref[...]
    h = jnp.maximum(h, 0.0) * _nrm(degs_ref[0, :, :1])
    o_ref[0] = h[:, :HID // 2]
    o_ref[1] = h[:, HID // 2:]


def _tc_mm1_body(a_ref, degs_ref, degd_ref, w1_ref, b1_ref, w2_ref, o_ref):
    agg = jnp.concatenate([a_ref[0], a_ref[1]], axis=-1)
    h = jnp.dot(agg, w1_ref[...], preferred_element_type=jnp.float32)
    h = h * _nrm(degd_ref[0, :, :1]) + b1_ref[...]
    h = jnp.maximum(h, 0.0) * _nrm(degs_ref[0, :, :1])
    m2 = jnp.dot(h, w2_ref[...], preferred_element_type=jnp.float32)
    o_ref[...] = jnp.concatenate(
        [m2, jnp.zeros((BM, D_IN - NCLS), jnp.float32)], axis=-1)


def _tc_post_body(a_ref, degd_ref, b_ref, o_ref):
    agg = a_ref[0, :, :NCLS] + a_ref[1, :, :NCLS]
    o_ref[...] = agg * _nrm(degd_ref[0, :, :1]) + b_ref[...]


def _row_spec(w):
    return pl.BlockSpec((BM, w), lambda i: (i, 0))


def _deg_spec(plane):
    return pl.BlockSpec((1, BM, DW), lambda i, _p=plane: (_p, i, 0))


def _half_spec(w):
    return pl.BlockSpec((NC, BM, w), lambda i: (0, i, 0))


def _full_spec(shape):
    nd = len(shape)
    return pl.BlockSpec(shape, lambda i, _n=nd: (0,) * _n)


_tc_pre = pl.pallas_call(
    _tc_pre_body,
    grid=(GRID,),
    in_specs=[_row_spec(D_IN), _deg_spec(0)],
    out_specs=_row_spec(D_IN),
    out_shape=jax.ShapeDtypeStruct((N, D_IN), jnp.float32),
)

_tc_mm0 = pl.pallas_call(
    _tc_mm0_body,
    grid=(GRID,),
    in_specs=[_half_spec(D_IN), _deg_spec(0), _deg_spec(1),
              _full_spec((D_IN, HID)), _full_spec((1, HID))],
    out_specs=_half_spec(HID // 2),
    out_shape=jax.ShapeDtypeStruct((NC, N, HID // 2), jnp.float32),
)

_tc_mm1 = pl.pallas_call(
    _tc_mm1_body,
    grid=(GRID,),
    in_specs=[_half_spec(HID // 2), _deg_spec(0), _deg_spec(1),
              _full_spec((HID, HID)), _full_spec((1, HID)),
              _full_spec((HID, NCLS))],
    out_specs=_row_spec(D_IN),
    out_shape=jax.ShapeDtypeStruct((N, D_IN), jnp.float32),
)

_tc_post = pl.pallas_call(
    _tc_post_body,
    grid=(GRID,),
    in_specs=[_half_spec(D_IN), _deg_spec(1), _full_spec((1, NCLS))],
    out_specs=_row_spec(NCLS),
    out_shape=jax.ShapeDtypeStruct((N, NCLS), jnp.float32),
)


def kernel(inputs, edge_index, W0, b0, W1, b1, W2, b2):
    src = edge_index[0]
    dst = edge_index[1]
    idx_r = edge_index.reshape(2, NS, E // NS // K, K)
    src32 = src.reshape(NC * NS, E // (NC * NS) // K, K)
    dst32 = dst.reshape(NC * NS, E // (NC * NS) // K, K)
    src16 = src.reshape(NS, E // NS // K, K)
    dst16 = dst.reshape(NS, E // NS // K, K)

    zeros_128 = jnp.zeros((KZ, D_IN), jnp.float32)
    ones_128 = jnp.ones((KZ, DW), jnp.float32)

    deg = _deg_call(idx_r, zeros_128, ones_128)

    xs = _tc_pre(inputs, deg)
    p0 = _agg_e128(xs, src32, dst32, zeros_128)
    h2 = _tc_mm0(p0, deg, deg, W0, b0.reshape(1, HID))
    a1 = _agg_c128(h2, src16, dst16, zeros_128)
    m2 = _tc_mm1(a1, deg, deg, W1, b1.reshape(1, HID), W2)
    p2 = _agg_e128(m2, src32, dst32, zeros_128)
    out = _tc_post(p2, deg, b2.reshape(1, NCLS))
    return out
